# baseline (speedup 1.0000x reference)
import functools

import jax
import jax.numpy as jnp
from jax import lax
from jax.experimental import pallas as pl
from jax.experimental.pallas import tpu as pltpu
from jax.experimental.pallas import tpu_sc as plsc

_NC = 2
_NS = 16
_NW = _NC * _NS
_CHUNK = 128


@functools.lru_cache(maxsize=None)
def _make_gather(vocab, dim, n_idx):
    assert n_idx % (_NW * _CHUNK) == 0
    b_per_w = n_idx // _NW
    n_chunks = b_per_w // _CHUNK
    mesh = plsc.VectorSubcoreMesh(core_axis_name="c", subcore_axis_name="s")

    @functools.partial(
        pl.kernel,
        out_type=jax.ShapeDtypeStruct((n_idx, dim), jnp.float32),
        mesh=mesh,
        scratch_types=[
            pltpu.VMEM((n_chunks, _CHUNK), jnp.int32),
            pltpu.VMEM((_CHUNK, dim), jnp.float32),
            pltpu.SemaphoreType.DMA,
        ],
        compiler_params=pltpu.CompilerParams(use_tc_tiling_on_sc=False),
    )
    def gather_kernel(idx_hbm, table_hbm, out_hbm, idx_v, rows_v, sem):
        wid = lax.axis_index("s") * _NC + lax.axis_index("c")
        base_chunk = wid * n_chunks
        pltpu.sync_copy(idx_hbm.at[pl.ds(base_chunk, n_chunks)], idx_v)

        def body(j, carry):
            pltpu.async_copy(table_hbm.at[idx_v.at[j]], rows_v, sem).wait()
            out_row = wid * b_per_w + j * _CHUNK
            pltpu.sync_copy(rows_v, out_hbm.at[pl.ds(out_row, _CHUNK)])
            return carry

        lax.fori_loop(0, n_chunks, body, 0)

    return gather_kernel


def kernel(x, embeddings):
    batch, seq = x.shape
    vocab, dim = embeddings.shape
    n_idx = batch * seq
    idx = x.reshape(n_idx // _CHUNK, _CHUNK).astype(jnp.int32)
    out = _make_gather(vocab, dim, n_idx)(idx, embeddings)
    return out.reshape(batch, seq, dim)


# speedup vs baseline: 1.1175x; 1.1175x over previous
import functools

import jax
import jax.numpy as jnp
from jax import lax
from jax.experimental import pallas as pl
from jax.experimental.pallas import tpu as pltpu
from jax.experimental.pallas import tpu_sc as plsc

_NC = 2
_NS = 16
_NW = _NC * _NS
_CHUNK = 128


_K = 4
_GROUP = _K * _CHUNK
_NBUF = 2


@functools.lru_cache(maxsize=None)
def _make_gather(vocab, dim, n_idx):
    assert n_idx % (_NW * _GROUP * _NBUF) == 0
    b_per_w = n_idx // _NW
    n_chunks = b_per_w // _CHUNK
    n_groups = n_chunks // _K
    mesh = plsc.VectorSubcoreMesh(core_axis_name="c", subcore_axis_name="s")

    @functools.partial(
        pl.kernel,
        out_type=jax.ShapeDtypeStruct((n_idx, dim), jnp.float32),
        mesh=mesh,
        scratch_types=[
            pltpu.VMEM((n_chunks, _CHUNK), jnp.int32),
            pltpu.VMEM((_NBUF, _GROUP, dim), jnp.float32),
            pltpu.SemaphoreType.DMA,
            pltpu.SemaphoreType.DMA,
        ],
        compiler_params=pltpu.CompilerParams(use_tc_tiling_on_sc=False),
    )
    def gather_kernel(idx_hbm, table_hbm, out_hbm, idx_v, rows_v, gsem, wsem):
        wid = lax.axis_index("s") * _NC + lax.axis_index("c")
        base_chunk = wid * n_chunks
        pltpu.sync_copy(idx_hbm.at[pl.ds(base_chunk, n_chunks)], idx_v)

        @pl.loop(0, n_groups, step=_NBUF)
        def _group(g0):
            for h in range(_NBUF):
                g = g0 + h

                @pl.when(g0 > 0)
                def _():
                    pltpu.make_async_copy(
                        rows_v.at[h], out_hbm.at[pl.ds(0, _GROUP)], wsem
                    ).wait()

                descs = [
                    pltpu.async_copy(
                        table_hbm.at[idx_v.at[g * _K + t]],
                        rows_v.at[h].at[pl.ds(t * _CHUNK, _CHUNK)],
                        gsem,
                    )
                    for t in range(_K)
                ]
                for d in descs:
                    d.wait()
                out_row = wid * b_per_w + g * _GROUP
                pltpu.async_copy(rows_v.at[h], out_hbm.at[pl.ds(out_row, _GROUP)], wsem)

        for h in range(_NBUF):
            pltpu.make_async_copy(
                rows_v.at[h], out_hbm.at[pl.ds(0, _GROUP)], wsem
            ).wait()

    return gather_kernel


def kernel(x, embeddings):
    batch, seq = x.shape
    vocab, dim = embeddings.shape
    n_idx = batch * seq
    idx = x.reshape(n_idx // _CHUNK, _CHUNK).astype(jnp.int32)
    out = _make_gather(vocab, dim, n_idx)(idx, embeddings)
    return out.reshape(batch, seq, dim)
